# baseline (speedup 1.0000x reference)
import functools

import jax
import jax.numpy as jnp
from jax import lax
from jax.experimental import pallas as pl
from jax.experimental.pallas import tpu as pltpu
from jax.experimental.pallas import tpu_sc as plsc

_NC = 2
_NS = 16
_NW = _NC * _NS
_CH = 128
_NBUF = 5


def _mm_body(wt_ref, x_ref, b_ref, o_ref):
    o_ref[...] = (
        lax.dot_general(
            wt_ref[...], x_ref[...],
            (((1,), (1,)), ((), ())),
            preferred_element_type=jnp.float32,
        )
        + b_ref[:, 0:1]
    )


def _matmul_t(Wt, x, bb):
    N, K = Wt.shape
    M = x.shape[0]
    BM = 2048
    return pl.pallas_call(
        _mm_body,
        grid=(pl.cdiv(M, BM),),
        in_specs=[
            pl.BlockSpec((N, K), lambda i: (0, 0)),
            pl.BlockSpec((BM, K), lambda i: (i, 0)),
            pl.BlockSpec((N, 128), lambda i: (0, 0)),
        ],
        out_specs=pl.BlockSpec((N, BM), lambda i: (0, i)),
        out_shape=jax.ShapeDtypeStruct((N, M), jnp.float32),
    )(Wt, x, bb)


def _tr_body(x_ref, eye_ref, o_ref):
    xt = lax.dot_general(
        x_ref[...], eye_ref[...],
        (((0,), (0,)), ((), ())),
        precision=lax.Precision.HIGHEST,
        preferred_element_type=jnp.float32,
    )
    h = xt.shape[0] // 2
    o_ref[:, 0:64] = xt[0:h, :]
    o_ref[:, 64:128] = xt[h:, :]


def _transpose_table(tt):
    D, V = tt.shape
    BC = 512
    nb = pl.cdiv(V, BC)
    return pl.pallas_call(
        _tr_body,
        grid=(nb,),
        in_specs=[
            pl.BlockSpec((D, BC), lambda i: (0, i)),
            pl.BlockSpec((D, D), lambda i: (0, 0)),
        ],
        out_specs=pl.BlockSpec((BC // 2, 2 * D), lambda i: (i, 0)),
        out_shape=jax.ShapeDtypeStruct((nb * BC // 2, 2 * D), jnp.float32),
    )(tt, jnp.eye(D, dtype=jnp.float32))


@functools.cache
def _make_gather(B, NCH, D):
    mesh = plsc.VectorSubcoreMesh(
        core_axis_name="c", subcore_axis_name="s",
        num_cores=_NC, num_subcores=_NS,
    )
    per_w = NCH * _CH
    tail = B % _CH
    rounds = NCH // _NBUF

    @functools.partial(
        pl.kernel,
        mesh=mesh,
        compiler_params=pltpu.CompilerParams(use_tc_tiling_on_sc=False),
        out_type=jax.ShapeDtypeStruct((B, D), jnp.float32),
        scratch_types=[
            pltpu.VMEM((NCH, _CH), jnp.int32),
            pltpu.VMEM((_NBUF, _CH, D), jnp.float32),
            pltpu.SemaphoreType.DMA((_NBUF,)),
            pltpu.SemaphoreType.DMA((_NBUF,)),
        ],
    )
    def gather_k(idx_hbm, table_hbm, out_hbm, idx_v, rows_v, gsem, osem):
        wid = lax.axis_index("s") * _NC + lax.axis_index("c")
        base = wid * per_w
        pltpu.sync_copy(idx_hbm.at[wid], idx_v)

        def store_copy(j, b, nrows):
            start = base + j * _CH
            return pltpu.make_async_copy(
                rows_v.at[b, pl.ds(0, nrows)],
                out_hbm.at[pl.ds(start, nrows)],
                osem.at[b],
            )

        def round_body(it, carry):
            j0 = it * _NBUF
            for b in range(_NBUF):
                j = j0 + b
                jp = j - _NBUF
                startp = base + jp * _CH

                @pl.when((it > 0) & (startp + _CH <= B))
                def _():
                    store_copy(jp, b, _CH).wait()

                if tail:
                    @pl.when((it > 0) & (startp < B) & (startp + _CH > B))
                    def _():
                        store_copy(jp, b, tail).wait()

                pltpu.async_copy(
                    table_hbm.at[idx_v.at[j]], rows_v.at[b], gsem.at[b]
                )
            for b in range(_NBUF):
                j = j0 + b
                start = base + j * _CH
                pltpu.make_async_copy(
                    table_hbm.at[idx_v.at[j]], rows_v.at[b], gsem.at[b]
                ).wait()

                @pl.when(start + _CH <= B)
                def _():
                    store_copy(j, b, _CH).start()

                if tail:
                    @pl.when((start < B) & (start + _CH > B))
                    def _():
                        store_copy(j, b, tail).start()

            return carry

        lax.fori_loop(0, rounds, round_body, 0)

        for b in range(_NBUF):
            j = (rounds - 1) * _NBUF + b
            start = base + j * _CH

            @pl.when(start + _CH <= B)
            def _():
                store_copy(j, b, _CH).wait()

            if tail:
                @pl.when((start < B) & (start + _CH > B))
                def _():
                    store_copy(j, b, tail).wait()

    return gather_k


def kernel(x_author, n_id_paper, W, b, emb_table):
    N = W.shape[1]
    bb = jnp.broadcast_to(b.reshape(N, 1), (N, 128))
    x_out = _matmul_t(W.T, x_author, bb).T

    B = n_id_paper.shape[0]
    D = emb_table.shape[1]
    per_call = _NW * _CH
    group = per_call * _NBUF
    Bpad = -(-B // group) * group
    pad_idx = jnp.arange(Bpad - B, dtype=jnp.int32)
    idx = jnp.concatenate([n_id_paper, pad_idx])
    idx = (idx & ~jnp.int32(511)) + 2 * (idx & 255) + ((idx >> 8) & 1)
    table_pair = _transpose_table(emb_table.T)
    table_rm = table_pair.reshape(-1, D)
    dep = jnp.where(jnp.isnan(table_pair[0, 0]), 1, 0).astype(jnp.int32)
    idx = (idx + dep).reshape(_NW, Bpad // per_call, _CH)
    emb_out = _make_gather(B, Bpad // per_call, D)(idx, table_rm)
    return (x_out, emb_out)


# speedup vs baseline: 2.5412x; 2.5412x over previous
import functools

import jax
import jax.numpy as jnp
from jax import lax
from jax.experimental import pallas as pl
from jax.experimental.pallas import tpu as pltpu
from jax.experimental.pallas import tpu_sc as plsc

_NC = 2
_NS = 16
_NW = _NC * _NS
_CH = 128
_NBUF = 5


def _mm_body(wt_ref, x_ref, b_ref, o_ref):
    o_ref[...] = (
        lax.dot_general(
            wt_ref[...], x_ref[...],
            (((1,), (1,)), ((), ())),
            preferred_element_type=jnp.float32,
        )
        + b_ref[:, 0:1]
    )


def _matmul_t(Wt, x, bb):
    N, K = Wt.shape
    M = x.shape[0]
    BM = 2048
    return pl.pallas_call(
        _mm_body,
        grid=(pl.cdiv(M, BM),),
        in_specs=[
            pl.BlockSpec((N, K), lambda i: (0, 0)),
            pl.BlockSpec((BM, K), lambda i: (i, 0)),
            pl.BlockSpec((N, 128), lambda i: (0, 0)),
        ],
        out_specs=pl.BlockSpec((N, BM), lambda i: (0, i)),
        out_shape=jax.ShapeDtypeStruct((N, M), jnp.float32),
    )(Wt, x, bb)


def _tr_body(x_ref, o_ref):
    BC = x_ref.shape[1]
    h = BC // 2
    SUB = 256
    for s in range(0, h, SUB):
        o_ref[s:s + SUB, 0:64] = x_ref[:, s:s + SUB].T
        o_ref[s:s + SUB, 64:128] = x_ref[:, h + s:h + s + SUB].T


def _transpose_table(tt):
    D, V = tt.shape
    BC = 2048
    nb = pl.cdiv(V, BC)
    return pl.pallas_call(
        _tr_body,
        grid=(nb,),
        in_specs=[
            pl.BlockSpec((D, BC), lambda i: (0, i)),
        ],
        out_specs=pl.BlockSpec((BC // 2, 2 * D), lambda i: (i, 0)),
        out_shape=jax.ShapeDtypeStruct((nb * BC // 2, 2 * D), jnp.float32),
    )(tt)


@functools.cache
def _make_gather(B, NCH, D):
    mesh = plsc.VectorSubcoreMesh(
        core_axis_name="c", subcore_axis_name="s",
        num_cores=_NC, num_subcores=_NS,
    )
    per_w = NCH * _CH
    tail = B % _CH
    rounds = NCH // _NBUF

    @functools.partial(
        pl.kernel,
        mesh=mesh,
        compiler_params=pltpu.CompilerParams(use_tc_tiling_on_sc=False),
        out_type=jax.ShapeDtypeStruct((B, D), jnp.float32),
        scratch_types=[
            pltpu.VMEM((NCH, _CH), jnp.int32),
            pltpu.VMEM((_NBUF, _CH, D), jnp.float32),
            pltpu.SemaphoreType.DMA((_NBUF,)),
            pltpu.SemaphoreType.DMA((_NBUF,)),
        ],
    )
    def gather_k(idx_hbm, table_hbm, out_hbm, idx_v, rows_v, gsem, osem):
        wid = lax.axis_index("s") * _NC + lax.axis_index("c")
        base = wid * per_w
        pltpu.sync_copy(idx_hbm.at[wid], idx_v)

        def store_copy(j, b, nrows):
            start = base + j * _CH
            return pltpu.make_async_copy(
                rows_v.at[b, pl.ds(0, nrows)],
                out_hbm.at[pl.ds(start, nrows)],
                osem.at[b],
            )

        def round_body(it, carry):
            j0 = it * _NBUF
            for b in range(_NBUF):
                j = j0 + b
                jp = j - _NBUF
                startp = base + jp * _CH

                @pl.when((it > 0) & (startp + _CH <= B))
                def _():
                    store_copy(jp, b, _CH).wait()

                if tail:
                    @pl.when((it > 0) & (startp < B) & (startp + _CH > B))
                    def _():
                        store_copy(jp, b, tail).wait()

                pltpu.async_copy(
                    table_hbm.at[idx_v.at[j]], rows_v.at[b], gsem.at[b]
                )
            for b in range(_NBUF):
                j = j0 + b
                start = base + j * _CH
                pltpu.make_async_copy(
                    table_hbm.at[idx_v.at[j]], rows_v.at[b], gsem.at[b]
                ).wait()

                @pl.when(start + _CH <= B)
                def _():
                    store_copy(j, b, _CH).start()

                if tail:
                    @pl.when((start < B) & (start + _CH > B))
                    def _():
                        store_copy(j, b, tail).start()

            return carry

        lax.fori_loop(0, rounds, round_body, 0)

        for b in range(_NBUF):
            j = (rounds - 1) * _NBUF + b
            start = base + j * _CH

            @pl.when(start + _CH <= B)
            def _():
                store_copy(j, b, _CH).wait()

            if tail:
                @pl.when((start < B) & (start + _CH > B))
                def _():
                    store_copy(j, b, tail).wait()

    return gather_k


def kernel(x_author, n_id_paper, W, b, emb_table):
    N = W.shape[1]
    bb = jnp.broadcast_to(b.reshape(N, 1), (N, 128))
    x_out = _matmul_t(W.T, x_author, bb).T

    B = n_id_paper.shape[0]
    D = emb_table.shape[1]
    per_call = _NW * _CH
    group = per_call * _NBUF
    Bpad = -(-B // group) * group
    pad_idx = jnp.arange(Bpad - B, dtype=jnp.int32)
    idx = jnp.concatenate([n_id_paper, pad_idx])
    idx = (idx & ~jnp.int32(511)) + 2 * (idx & 255) + ((idx >> 8) & 1)
    table_pair = _transpose_table(emb_table.T)
    table_rm = table_pair.reshape(-1, D)
    dep = jnp.where(jnp.isnan(table_pair[0, 0]), 1, 0).astype(jnp.int32)
    idx = (idx + dep).reshape(_NW, Bpad // per_call, _CH)
    emb_out = _make_gather(B, Bpad // per_call, D)(idx, table_rm)
    return (x_out, emb_out)


# speedup vs baseline: 2.5420x; 1.0003x over previous
import functools

import jax
import jax.numpy as jnp
from jax import lax
from jax.experimental import pallas as pl
from jax.experimental.pallas import tpu as pltpu
from jax.experimental.pallas import tpu_sc as plsc

_NC = 2
_NS = 16
_NW = _NC * _NS
_CH = 128
_NBUF = 5
_BC = 2048


def _mm_body(wt_ref, x_ref, b_ref, o_ref):
    o_ref[...] = (
        lax.dot_general(
            wt_ref[...], x_ref[...],
            (((1,), (1,)), ((), ())),
            preferred_element_type=jnp.float32,
        )
        + b_ref[:, 0:1]
    )


def _matmul_t(Wt, x, bb):
    N, K = Wt.shape
    M = x.shape[0]
    BM = 2048
    return pl.pallas_call(
        _mm_body,
        grid=(pl.cdiv(M, BM),),
        in_specs=[
            pl.BlockSpec((N, K), lambda i: (0, 0)),
            pl.BlockSpec((BM, K), lambda i: (i, 0)),
            pl.BlockSpec((N, 128), lambda i: (0, 0)),
        ],
        out_specs=pl.BlockSpec((N, BM), lambda i: (0, i)),
        out_shape=jax.ShapeDtypeStruct((N, M), jnp.float32),
    )(Wt, x, bb)


def _tr_body(x_ref, o_ref):
    BC = x_ref.shape[1]
    h = BC // 2
    SUB = 256
    for s in range(0, h, SUB):
        o_ref[s:s + SUB, 0:64] = x_ref[:, s:s + SUB].T
        o_ref[s:s + SUB, 64:128] = x_ref[:, h + s:h + s + SUB].T


def _transpose_table(tt):
    D, V = tt.shape
    BC = _BC
    nb = pl.cdiv(V, BC)
    return pl.pallas_call(
        _tr_body,
        grid=(nb,),
        in_specs=[
            pl.BlockSpec((D, BC), lambda i: (0, i)),
        ],
        out_specs=pl.BlockSpec((BC // 2, 2 * D), lambda i: (i, 0)),
        out_shape=jax.ShapeDtypeStruct((nb * BC // 2, 2 * D), jnp.float32),
    )(tt)


@functools.cache
def _make_gather(B, NCH, D):
    mesh = plsc.VectorSubcoreMesh(
        core_axis_name="c", subcore_axis_name="s",
        num_cores=_NC, num_subcores=_NS,
    )
    per_w = NCH * _CH
    tail = B % _CH
    rounds = NCH // _NBUF

    @functools.partial(
        pl.kernel,
        mesh=mesh,
        compiler_params=pltpu.CompilerParams(use_tc_tiling_on_sc=False),
        out_type=jax.ShapeDtypeStruct((B, D), jnp.float32),
        scratch_types=[
            pltpu.VMEM((NCH, _CH), jnp.int32),
            pltpu.VMEM((_NBUF, _CH, D), jnp.float32),
            pltpu.SemaphoreType.DMA((_NBUF,)),
            pltpu.SemaphoreType.DMA((_NBUF,)),
        ],
    )
    def gather_k(idx_hbm, table_hbm, out_hbm, idx_v, rows_v, gsem, osem):
        wid = lax.axis_index("s") * _NC + lax.axis_index("c")
        base = wid * per_w
        pltpu.sync_copy(idx_hbm.at[wid], idx_v)

        def store_copy(j, b, nrows):
            start = base + j * _CH
            return pltpu.make_async_copy(
                rows_v.at[b, pl.ds(0, nrows)],
                out_hbm.at[pl.ds(start, nrows)],
                osem.at[b],
            )

        def round_body(it, carry):
            j0 = it * _NBUF
            for b in range(_NBUF):
                j = j0 + b
                jp = j - _NBUF
                startp = base + jp * _CH

                @pl.when((it > 0) & (startp + _CH <= B))
                def _():
                    store_copy(jp, b, _CH).wait()

                if tail:
                    @pl.when((it > 0) & (startp < B) & (startp + _CH > B))
                    def _():
                        store_copy(jp, b, tail).wait()

                pltpu.async_copy(
                    table_hbm.at[idx_v.at[j]], rows_v.at[b], gsem.at[b]
                )
            for b in range(_NBUF):
                j = j0 + b
                start = base + j * _CH
                pltpu.make_async_copy(
                    table_hbm.at[idx_v.at[j]], rows_v.at[b], gsem.at[b]
                ).wait()

                @pl.when(start + _CH <= B)
                def _():
                    store_copy(j, b, _CH).start()

                if tail:
                    @pl.when((start < B) & (start + _CH > B))
                    def _():
                        store_copy(j, b, tail).start()

            return carry

        lax.fori_loop(0, rounds, round_body, 0)

        for b in range(_NBUF):
            j = (rounds - 1) * _NBUF + b
            start = base + j * _CH

            @pl.when(start + _CH <= B)
            def _():
                store_copy(j, b, _CH).wait()

            if tail:
                @pl.when((start < B) & (start + _CH > B))
                def _():
                    store_copy(j, b, tail).wait()

    return gather_k


def kernel(x_author, n_id_paper, W, b, emb_table):
    N = W.shape[1]
    bb = jnp.broadcast_to(b.reshape(N, 1), (N, 128))
    x_out = _matmul_t(W.T, x_author, bb).T

    B = n_id_paper.shape[0]
    D = emb_table.shape[1]
    per_call = _NW * _CH
    group = per_call * _NBUF
    Bpad = -(-B // group) * group
    pad_idx = jnp.arange(Bpad - B, dtype=jnp.int32)
    idx = jnp.concatenate([n_id_paper, pad_idx])
    half = _BC // 2
    idx = (idx & ~jnp.int32(_BC - 1)) + 2 * (idx & (half - 1)) + ((idx // half) & 1)
    table_pair = _transpose_table(emb_table.T)
    table_rm = table_pair.reshape(-1, D)
    dep = jnp.where(jnp.isnan(table_pair[0, 0]), 1, 0).astype(jnp.int32)
    idx = (idx + dep).reshape(_NW, Bpad // per_call, _CH)
    emb_out = _make_gather(B, Bpad // per_call, D)(idx, table_rm)
    return (x_out, emb_out)
